# baseline (speedup 1.0000x reference)
import jax
import jax.numpy as jnp
from jax.experimental import pallas as pl
from jax.experimental.pallas import tpu as pltpu

T, D, E, TOP_K = 4096, 768, 8, 2
BT = 2048
LANES = 128


def _moe_block(x_ref, wr_ref, br_ref, we_ref, be_ref, out_ref, loss_ref, acc_ref):
    pid = pl.program_id(0)
    nsteps = pl.num_programs(0)

    x = x_ref[...]
    logits = jnp.dot(x, wr_ref[...], preferred_element_type=jnp.float32)
    logits = logits + br_ref[...]
    lane = jax.lax.broadcasted_iota(jnp.int32, (BT, LANES), 1)
    valid = lane < E
    neg = jnp.float32(-1e30)
    logits = jnp.where(valid, logits, neg)

    m = jnp.max(logits, axis=-1, keepdims=True)
    ex = jnp.exp(logits - m)
    ex = jnp.where(valid, ex, 0.0)
    denom = jnp.sum(ex, axis=-1, keepdims=True)
    probs = ex / denom

    big = jnp.int32(10**9)
    m1 = jnp.max(probs, axis=-1, keepdims=True)
    i1 = jnp.min(jnp.where((probs == m1) & valid, lane, big), axis=-1, keepdims=True)
    probs2 = jnp.where(lane == i1, neg, probs)
    m2 = jnp.max(probs2, axis=-1, keepdims=True)
    i2 = jnp.min(jnp.where((probs2 == m2) & valid, lane, big), axis=-1, keepdims=True)

    oh1 = (lane == i1).astype(jnp.float32)
    oh2 = (lane == i2).astype(jnp.float32)
    wsum = m1 + m2
    gate = (m1 / wsum) * oh1 + (m2 / wsum) * oh2

    part = jnp.sum(oh1 + oh2, axis=0, keepdims=True)
    psum = jnp.sum(probs, axis=0, keepdims=True)

    @pl.when(pid == 0)
    def _init():
        acc_ref[...] = jnp.zeros_like(acc_ref)

    acc_ref[0:1, :] += part
    acc_ref[1:2, :] += psum

    xb = x.astype(jnp.bfloat16)
    acc = jnp.zeros((BT, D), dtype=jnp.float32)
    for e in range(E):
        ye = jnp.dot(xb, we_ref[e], preferred_element_type=jnp.float32)
        ye = ye + be_ref[e][None, :]
        acc = acc + gate[:, e][:, None] * ye
    out_ref[...] = acc

    @pl.when(pid == nsteps - 1)
    def _fin():
        f = acc_ref[0:1, :] / jnp.float32(T)
        p = acc_ref[1:2, :] / jnp.float32(T)
        loss_ref[...] = jnp.sum(jnp.float32(E) * f * p).reshape(1, 1)


def kernel(x, Wr, br, We, be):
    wr_pad = jnp.zeros((D, LANES), jnp.float32).at[:, :E].set(Wr)
    br_pad = jnp.zeros((1, LANES), jnp.float32).at[0, :E].set(br)
    We = We.astype(jnp.bfloat16)

    grid = (T // BT,)
    out, loss = pl.pallas_call(
        _moe_block,
        grid=grid,
        in_specs=[
            pl.BlockSpec((BT, D), lambda i: (i, 0)),
            pl.BlockSpec((D, LANES), lambda i: (0, 0)),
            pl.BlockSpec((1, LANES), lambda i: (0, 0)),
            pl.BlockSpec((E, D, D), lambda i: (0, 0, 0)),
            pl.BlockSpec((E, D), lambda i: (0, 0)),
        ],
        out_specs=[
            pl.BlockSpec((BT, D), lambda i: (i, 0)),
            pl.BlockSpec((1, 1), lambda i: (0, 0)),
        ],
        out_shape=[
            jax.ShapeDtypeStruct((T, D), jnp.float32),
            jax.ShapeDtypeStruct((1, 1), jnp.float32),
        ],
        scratch_shapes=[pltpu.VMEM((2, LANES), jnp.float32)],
    )(x, wr_pad, br_pad, We, be)
    return out, loss.reshape(())


# speedup vs baseline: 1.1471x; 1.1471x over previous
import jax
import jax.numpy as jnp
from jax.experimental import pallas as pl
from jax.experimental.pallas import tpu as pltpu

T, D, E, TOP_K = 4096, 768, 8, 2
BT = 1024
LANES = 128


def _moe_block(x_ref, wr_ref, br_ref, we_ref, be_ref, out_ref, loss_ref, acc_ref):
    pid = pl.program_id(0)
    nsteps = pl.num_programs(0)

    x = x_ref[...]
    logits = jnp.dot(x, wr_ref[...], preferred_element_type=jnp.float32)
    logits = logits + br_ref[...]
    lane = jax.lax.broadcasted_iota(jnp.int32, (BT, LANES), 1)
    valid = lane < E
    neg = jnp.float32(-1e30)
    logits = jnp.where(valid, logits, neg)

    m = jnp.max(logits, axis=-1, keepdims=True)
    ex = jnp.exp(logits - m)
    ex = jnp.where(valid, ex, 0.0)
    denom = jnp.sum(ex, axis=-1, keepdims=True)
    probs = ex / denom

    big = jnp.int32(10**9)
    m1 = jnp.max(probs, axis=-1, keepdims=True)
    i1 = jnp.min(jnp.where((probs == m1) & valid, lane, big), axis=-1, keepdims=True)
    probs2 = jnp.where(lane == i1, neg, probs)
    m2 = jnp.max(probs2, axis=-1, keepdims=True)
    i2 = jnp.min(jnp.where((probs2 == m2) & valid, lane, big), axis=-1, keepdims=True)

    oh1 = (lane == i1).astype(jnp.float32)
    oh2 = (lane == i2).astype(jnp.float32)
    wsum = m1 + m2
    gate = (m1 / wsum) * oh1 + (m2 / wsum) * oh2

    part = jnp.sum(oh1 + oh2, axis=0, keepdims=True)
    psum = jnp.sum(probs, axis=0, keepdims=True)

    @pl.when(pid == 0)
    def _init():
        acc_ref[...] = jnp.zeros_like(acc_ref)

    acc_ref[0:1, :] += part
    acc_ref[1:2, :] += psum

    acc0 = jnp.zeros((BT, D), dtype=jnp.float32)
    acc1 = jnp.zeros((BT, D), dtype=jnp.float32)
    for e in range(E // 2):
        y0 = jnp.dot(x, we_ref[2 * e], preferred_element_type=jnp.float32)
        y0 = y0 + be_ref[2 * e][None, :]
        acc0 = acc0 + gate[:, 2 * e][:, None] * y0
        y1 = jnp.dot(x, we_ref[2 * e + 1], preferred_element_type=jnp.float32)
        y1 = y1 + be_ref[2 * e + 1][None, :]
        acc1 = acc1 + gate[:, 2 * e + 1][:, None] * y1
    out_ref[...] = acc0 + acc1

    @pl.when(pid == nsteps - 1)
    def _fin():
        f = acc_ref[0:1, :] / jnp.float32(T)
        p = acc_ref[1:2, :] / jnp.float32(T)
        loss_ref[...] = jnp.sum(jnp.float32(E) * f * p).reshape(1, 1)


def kernel(x, Wr, br, We, be):
    wr_pad = jnp.zeros((D, LANES), jnp.float32).at[:, :E].set(Wr)
    br_pad = jnp.zeros((1, LANES), jnp.float32).at[0, :E].set(br)

    grid = (T // BT,)
    out, loss = pl.pallas_call(
        _moe_block,
        grid=grid,
        in_specs=[
            pl.BlockSpec((BT, D), lambda i: (i, 0)),
            pl.BlockSpec((D, LANES), lambda i: (0, 0)),
            pl.BlockSpec((1, LANES), lambda i: (0, 0)),
            pl.BlockSpec((E, D, D), lambda i: (0, 0, 0)),
            pl.BlockSpec((E, D), lambda i: (0, 0)),
        ],
        out_specs=[
            pl.BlockSpec((BT, D), lambda i: (i, 0)),
            pl.BlockSpec((1, 1), lambda i: (0, 0)),
        ],
        out_shape=[
            jax.ShapeDtypeStruct((T, D), jnp.float32),
            jax.ShapeDtypeStruct((1, 1), jnp.float32),
        ],
        scratch_shapes=[pltpu.VMEM((2, LANES), jnp.float32)],
    )(x, wr_pad, br_pad, We, be)
    return out, loss.reshape(())


# speedup vs baseline: 1.1613x; 1.0124x over previous
import jax
import jax.numpy as jnp
from jax.experimental import pallas as pl
from jax.experimental.pallas import tpu as pltpu

T, D, E, TOP_K = 4096, 768, 8, 2
BT = 1024
LANES = 128


def _moe_block(x_ref, wr_ref, br_ref, we_ref, be_ref, out_ref, loss_ref, acc_ref):
    pid = pl.program_id(0)
    nsteps = pl.num_programs(0)

    x = x_ref[...]
    logits = jnp.dot(x, wr_ref[...], preferred_element_type=jnp.float32)
    logits = logits + br_ref[...]
    lane = jax.lax.broadcasted_iota(jnp.int32, (BT, LANES), 1)
    valid = lane < E
    neg = jnp.float32(-1e30)
    logits = jnp.where(valid, logits, neg)

    m = jnp.max(logits, axis=-1, keepdims=True)
    ex = jnp.exp(logits - m)
    ex = jnp.where(valid, ex, 0.0)
    denom = jnp.sum(ex, axis=-1, keepdims=True)
    probs = ex / denom

    big = jnp.int32(10**9)
    m1 = jnp.max(probs, axis=-1, keepdims=True)
    i1 = jnp.min(jnp.where((probs == m1) & valid, lane, big), axis=-1, keepdims=True)
    probs2 = jnp.where(lane == i1, neg, probs)
    m2 = jnp.max(probs2, axis=-1, keepdims=True)
    i2 = jnp.min(jnp.where((probs2 == m2) & valid, lane, big), axis=-1, keepdims=True)

    oh1 = (lane == i1).astype(jnp.float32)
    oh2 = (lane == i2).astype(jnp.float32)
    wsum = m1 + m2
    gate = (m1 / wsum) * oh1 + (m2 / wsum) * oh2

    part = jnp.sum(oh1 + oh2, axis=0, keepdims=True)
    psum = jnp.sum(probs, axis=0, keepdims=True)

    @pl.when(pid == 0)
    def _init():
        acc_ref[...] = jnp.zeros_like(acc_ref)

    acc_ref[0:1, :] += part
    acc_ref[1:2, :] += psum

    acc = jnp.zeros((BT, D), dtype=jnp.float32)
    for e in range(E):
        ye = jnp.dot(x, we_ref[e], preferred_element_type=jnp.float32)
        ye = ye + be_ref[e][None, :]
        acc = acc + gate[:, e][:, None] * ye
    out_ref[...] = acc

    @pl.when(pid == nsteps - 1)
    def _fin():
        f = acc_ref[0:1, :] / jnp.float32(T)
        p = acc_ref[1:2, :] / jnp.float32(T)
        loss_ref[...] = jnp.sum(jnp.float32(E) * f * p).reshape(1, 1)


def kernel(x, Wr, br, We, be):
    wr_pad = jnp.zeros((D, LANES), jnp.float32).at[:, :E].set(Wr)
    br_pad = jnp.zeros((1, LANES), jnp.float32).at[0, :E].set(br)

    grid = (T // BT,)
    out, loss = pl.pallas_call(
        _moe_block,
        grid=grid,
        in_specs=[
            pl.BlockSpec((BT, D), lambda i: (i, 0)),
            pl.BlockSpec((D, LANES), lambda i: (0, 0)),
            pl.BlockSpec((1, LANES), lambda i: (0, 0)),
            pl.BlockSpec((E, D, D), lambda i: (0, 0, 0)),
            pl.BlockSpec((E, D), lambda i: (0, 0)),
        ],
        out_specs=[
            pl.BlockSpec((BT, D), lambda i: (i, 0)),
            pl.BlockSpec((1, 1), lambda i: (0, 0)),
        ],
        out_shape=[
            jax.ShapeDtypeStruct((T, D), jnp.float32),
            jax.ShapeDtypeStruct((1, 1), jnp.float32),
        ],
        scratch_shapes=[pltpu.VMEM((2, LANES), jnp.float32)],
    )(x, wr_pad, br_pad, We, be)
    return out, loss.reshape(())
